# baseline (device time: 26546 ns/iter reference)
import jax
import jax.numpy as jnp
from jax import lax
from jax.experimental import pallas as pl
from jax.experimental.pallas import tpu as pltpu

N_DEV = 4
N_LAYERS = 3


def kernel(x, Win0, Wout0, Win1, Wout1, Win2, Wout2):
    b, d = x.shape
    h_per = Win0.shape[1]

    def body(x_ref, win0_ref, wout0_ref, win1_ref, wout1_ref,
             win2_ref, wout2_ref, out_ref,
             win_buf, wout_buf, own_ref, comm_ref,
             wsems, send_sems, recv_sems):
        my = lax.axis_index("i")
        win_hbm = [win0_ref, win1_ref, win2_ref]
        wout_hbm = [wout0_ref, wout1_ref, wout2_ref]

        wcopies = []
        for L in range(N_LAYERS):
            cin = pltpu.make_async_copy(win_hbm[L], win_buf.at[L], wsems.at[L, 0])
            cout = pltpu.make_async_copy(wout_hbm[L], wout_buf.at[L], wsems.at[L, 1])
            wcopies.append((cin, cout))
        wcopies[0][0].start()
        wcopies[0][1].start()

        barrier_sem = pltpu.get_barrier_semaphore()
        for off in range(1, N_DEV):
            pl.semaphore_signal(
                barrier_sem, inc=1,
                device_id=(lax.rem(my + off, N_DEV),),
                device_id_type=pl.DeviceIdType.MESH,
            )
        pl.semaphore_wait(barrier_sem, N_DEV - 1)

        wcopies[0][0].wait()
        wcopies[0][1].wait()
        xv = x_ref[:, :]
        for L in range(N_LAYERS):
            h = jnp.dot(xv, win_buf[L],
                        preferred_element_type=jnp.float32)
            h = jnp.maximum(h, 0.0)
            partial = jnp.dot(h, wout_buf[L],
                              preferred_element_type=jnp.float32)
            own_ref[L] = partial.astype(jnp.bfloat16)

            sends = []
            for off in range(1, N_DEV):
                tgt = lax.rem(my + off, N_DEV)
                rdma = pltpu.make_async_remote_copy(
                    src_ref=own_ref.at[L],
                    dst_ref=comm_ref.at[L, 3 - off],
                    send_sem=send_sems.at[L, off - 1],
                    recv_sem=recv_sems.at[L, 3 - off],
                    device_id=(tgt,),
                    device_id_type=pl.DeviceIdType.MESH,
                )
                rdma.start()
                sends.append(rdma)

            if L + 1 < N_LAYERS:
                wcopies[L + 1][0].start()
                wcopies[L + 1][1].start()
                wcopies[L + 1][0].wait()
                wcopies[L + 1][1].wait()

            for j in range(N_DEV - 1):
                recv = pltpu.make_async_remote_copy(
                    src_ref=own_ref.at[L],
                    dst_ref=comm_ref.at[L, j],
                    send_sem=send_sems.at[L, j],
                    recv_sem=recv_sems.at[L, j],
                    device_id=(my,),
                    device_id_type=pl.DeviceIdType.MESH,
                )
                recv.wait_recv()
            for rdma in sends:
                rdma.wait_send()

            acc = (partial
                   + comm_ref[L, 0].astype(jnp.float32)
                   + comm_ref[L, 1].astype(jnp.float32)
                   + comm_ref[L, 2].astype(jnp.float32))
            if L < N_LAYERS - 1:
                xv = acc
            else:
                out_ref[:, :] = acc

    return pl.pallas_call(
        body,
        out_shape=jax.ShapeDtypeStruct((b, d), jnp.float32),
        in_specs=[pl.BlockSpec(memory_space=pltpu.VMEM)]
        + [pl.BlockSpec(memory_space=pltpu.MemorySpace.HBM)] * 6,
        out_specs=pl.BlockSpec(memory_space=pltpu.VMEM),
        scratch_shapes=[
            pltpu.VMEM((N_LAYERS, d, h_per), jnp.float32),
            pltpu.VMEM((N_LAYERS, h_per, d), jnp.float32),
            pltpu.VMEM((N_LAYERS, b, d), jnp.bfloat16),
            pltpu.VMEM((N_LAYERS, N_DEV - 1, b, d), jnp.bfloat16),
            pltpu.SemaphoreType.DMA((N_LAYERS, 2)),
            pltpu.SemaphoreType.DMA((N_LAYERS, N_DEV - 1)),
            pltpu.SemaphoreType.DMA((N_LAYERS, N_DEV - 1)),
        ],
        compiler_params=pltpu.CompilerParams(collective_id=0),
    )(x, Win0, Wout0, Win1, Wout1, Win2, Wout2)


# device time: 24089 ns/iter; 1.1020x vs baseline; 1.1020x over previous
import jax
import jax.numpy as jnp
from jax import lax
from jax.experimental import pallas as pl
from jax.experimental.pallas import tpu as pltpu

N_DEV = 4
N_LAYERS = 3


def kernel(x, Win0, Wout0, Win1, Wout1, Win2, Wout2):
    b, d = x.shape

    def body(x_ref, win0_ref, wout0_ref, win1_ref, wout1_ref,
             win2_ref, wout2_ref, out_ref,
             own_ref, comm_ref, send_sems, recv_sems):
        my = lax.axis_index("i")
        win_refs = [win0_ref, win1_ref, win2_ref]
        wout_refs = [wout0_ref, wout1_ref, wout2_ref]

        barrier_sem = pltpu.get_barrier_semaphore()
        pl.semaphore_signal(barrier_sem, inc=1, device_id=(my,),
                            device_id_type=pl.DeviceIdType.MESH)
        pl.semaphore_wait(barrier_sem, 1)

        x_bf = x_ref[:, :].astype(jnp.bfloat16)
        for L in range(N_LAYERS):
            h = jnp.dot(x_bf, win_refs[L][:, :],
                        preferred_element_type=jnp.float32)
            h = jnp.maximum(h, 0.0).astype(jnp.bfloat16)
            partial = jnp.dot(h, wout_refs[L][:, :],
                              preferred_element_type=jnp.float32)
            own_ref[L] = partial.astype(jnp.bfloat16)

            sends = []
            for off in range(1, N_DEV):
                tgt = lax.rem(my + off, N_DEV)
                rdma = pltpu.make_async_remote_copy(
                    src_ref=own_ref.at[L],
                    dst_ref=comm_ref.at[L, 3 - off],
                    send_sem=send_sems.at[L, off - 1],
                    recv_sem=recv_sems.at[L, 3 - off],
                    device_id=(tgt,),
                    device_id_type=pl.DeviceIdType.MESH,
                )
                rdma.start()
                sends.append(rdma)

            for j in range(N_DEV - 1):
                recv = pltpu.make_async_remote_copy(
                    src_ref=own_ref.at[L],
                    dst_ref=comm_ref.at[L, j],
                    send_sem=send_sems.at[L, j],
                    recv_sem=recv_sems.at[L, j],
                    device_id=(my,),
                    device_id_type=pl.DeviceIdType.MESH,
                )
                recv.wait_recv()
            for rdma in sends:
                rdma.wait_send()

            acc = (partial
                   + comm_ref[L, 0].astype(jnp.float32)
                   + comm_ref[L, 1].astype(jnp.float32)
                   + comm_ref[L, 2].astype(jnp.float32))
            if L < N_LAYERS - 1:
                x_bf = acc.astype(jnp.bfloat16)
            else:
                out_ref[:, :] = acc

    bf = jnp.bfloat16
    return pl.pallas_call(
        body,
        out_shape=jax.ShapeDtypeStruct((b, d), jnp.float32),
        in_specs=[pl.BlockSpec(memory_space=pltpu.VMEM)] * 7,
        out_specs=pl.BlockSpec(memory_space=pltpu.VMEM),
        scratch_shapes=[
            pltpu.VMEM((N_LAYERS, b, d), jnp.bfloat16),
            pltpu.VMEM((N_LAYERS, N_DEV - 1, b, d), jnp.bfloat16),
            pltpu.SemaphoreType.DMA((N_LAYERS, N_DEV - 1)),
            pltpu.SemaphoreType.DMA((N_LAYERS, N_DEV - 1)),
        ],
        compiler_params=pltpu.CompilerParams(collective_id=0),
    )(x, Win0.astype(bf), Wout0.astype(bf), Win1.astype(bf),
      Wout1.astype(bf), Win2.astype(bf), Wout2.astype(bf))


# device time: 23382 ns/iter; 1.1353x vs baseline; 1.0302x over previous
import jax
import jax.numpy as jnp
from jax import lax
from jax.experimental import pallas as pl
from jax.experimental.pallas import tpu as pltpu

N_DEV = 4
N_LAYERS = 3


def kernel(x, Win0, Wout0, Win1, Wout1, Win2, Wout2):
    b, d = x.shape

    def body(x_ref, win0_ref, wout0_ref, win1_ref, wout1_ref,
             win2_ref, wout2_ref, out_ref,
             own_ref, comm_ref, send_sems, recv_sems):
        my = lax.axis_index("i")
        win_refs = [win0_ref, win1_ref, win2_ref]
        wout_refs = [wout0_ref, wout1_ref, wout2_ref]

        barrier_sem = pltpu.get_barrier_semaphore()
        pl.semaphore_signal(barrier_sem, inc=1, device_id=(my,),
                            device_id_type=pl.DeviceIdType.MESH)
        pl.semaphore_wait(barrier_sem, 1)

        x_bf = x_ref[:, :].astype(jnp.bfloat16)
        all_sends = []
        for L in range(N_LAYERS):
            h = jnp.dot(x_bf, win_refs[L][:, :],
                        preferred_element_type=jnp.float32)
            h = jnp.maximum(h, 0.0).astype(jnp.bfloat16)
            partial = jnp.dot(h, wout_refs[L][:, :],
                              preferred_element_type=jnp.float32)
            own_ref[L] = partial.astype(jnp.bfloat16)

            sends = []
            for off in (2, 1, 3):
                tgt = lax.rem(my + off, N_DEV)
                rdma = pltpu.make_async_remote_copy(
                    src_ref=own_ref.at[L],
                    dst_ref=comm_ref.at[L, 3 - off],
                    send_sem=send_sems.at[L, off - 1],
                    recv_sem=recv_sems.at[L, 3 - off],
                    device_id=(tgt,),
                    device_id_type=pl.DeviceIdType.MESH,
                )
                rdma.start()
                sends.append(rdma)

            for j in range(N_DEV - 1):
                recv = pltpu.make_async_remote_copy(
                    src_ref=own_ref.at[L],
                    dst_ref=comm_ref.at[L, j],
                    send_sem=send_sems.at[L, j],
                    recv_sem=recv_sems.at[L, j],
                    device_id=(my,),
                    device_id_type=pl.DeviceIdType.MESH,
                )
                recv.wait_recv()
            all_sends.extend(sends)

            acc = (partial
                   + comm_ref[L, 0].astype(jnp.float32)
                   + comm_ref[L, 1].astype(jnp.float32)
                   + comm_ref[L, 2].astype(jnp.float32))
            if L < N_LAYERS - 1:
                x_bf = acc.astype(jnp.bfloat16)
            else:
                out_ref[:, :] = acc

        for rdma in all_sends:
            rdma.wait_send()

    bf = jnp.bfloat16
    return pl.pallas_call(
        body,
        out_shape=jax.ShapeDtypeStruct((b, d), jnp.float32),
        in_specs=[pl.BlockSpec(memory_space=pltpu.VMEM)] * 7,
        out_specs=pl.BlockSpec(memory_space=pltpu.VMEM),
        scratch_shapes=[
            pltpu.VMEM((N_LAYERS, b, d), jnp.bfloat16),
            pltpu.VMEM((N_LAYERS, N_DEV - 1, b, d), jnp.bfloat16),
            pltpu.SemaphoreType.DMA((N_LAYERS, N_DEV - 1)),
            pltpu.SemaphoreType.DMA((N_LAYERS, N_DEV - 1)),
        ],
        compiler_params=pltpu.CompilerParams(collective_id=0),
    )(x, Win0.astype(bf), Wout0.astype(bf), Win1.astype(bf),
      Wout1.astype(bf), Win2.astype(bf), Wout2.astype(bf))
